# baseline (speedup 1.0000x reference)
import functools

import jax
import jax.numpy as jnp
from jax import lax
from jax.experimental import pallas as pl
from jax.experimental.pallas import tpu as pltpu
from jax.experimental.pallas import tpu_sc as plsc

_B, _S, _D = 4, 8192, 2048
_W = _S // 4
_LANES = 16
_CHUNK = 8
_BIG = 1 << 30

_mesh = plsc.VectorSubcoreMesh(core_axis_name="c", subcore_axis_name="s")


@functools.partial(
    pl.kernel,
    out_type=jax.ShapeDtypeStruct((_B, _D), jnp.float32),
    mesh=_mesh,
    scratch_types=[
        pltpu.VMEM((_W,), jnp.int32),
        pltpu.VMEM((_D,), jnp.float32),
    ],
)
def _extract_eos(tokens_hbm, words_hbm, out_hbm, words_v, row_v):
    num_c = lax.axis_size("c")
    wid = lax.axis_index("s") * num_c + lax.axis_index("c")

    @pl.when(wid < _B)
    def _():
        b = wid
        pltpu.sync_copy(words_hbm.at[b], words_v)
        iota = lax.iota(jnp.int32, _LANES)
        n_iters = _W // (_LANES * _CHUNK)

        def body(j, cand):
            base = j * (_LANES * _CHUNK)
            for k in range(_CHUNK):
                off = base + k * _LANES
                v = words_v[pl.ds(off, _LANES)]
                b0 = (v & 0x000000FF) != 0
                b1 = (v & 0x0000FF00) != 0
                b2 = (v & 0x00FF0000) != 0
                sub = jnp.where(b0, 0, jnp.where(b1, 1, jnp.where(b2, 2, 3)))
                pos = (iota + off) * 4 + sub.astype(jnp.int32)
                cand = jnp.minimum(cand, jnp.where(v != 0, pos, _BIG))
            return cand

        cand0 = jnp.full((_LANES,), _BIG, jnp.int32)
        cand = lax.fori_loop(0, n_iters, body, cand0)
        for sh in (8, 4, 2, 1):
            rot = lax.rem(iota + sh, jnp.full((_LANES,), _LANES, jnp.int32))
            cand = jnp.minimum(
                cand, cand.at[rot].get(mode="promise_in_bounds")
            )
        m = cand[0]
        idx = jnp.where(m < _BIG, m, jnp.int32(0))
        pltpu.sync_copy(tokens_hbm.at[b, idx], row_v)
        pltpu.sync_copy(row_v, out_hbm.at[b])


def kernel(tokens, eos_token_mask):
    words = lax.bitcast_convert_type(
        eos_token_mask.reshape(_B, _W, 4).view(jnp.uint8), jnp.int32
    )
    return _extract_eos(tokens, words)
